# Initial kernel scaffold; baseline (speedup 1.0000x reference)
import jax
import jax.numpy as jnp
from jax.experimental import pallas as pl


def kernel(x, edge_index, batch, params, fc_W, fc_b):
    raise NotImplementedError("write your pallas kernel here")


# speedup vs baseline: 7.5665x; 7.5665x over previous
import functools

import jax
import jax.numpy as jnp
from jax import lax
from jax.experimental import pallas as pl
from jax.experimental.pallas import tpu as pltpu
from jax.experimental.pallas import tpu_sc as plsc

_N = 10000
_E = 320000
_F = 128
_NG = 64
_NCLASS = 10

_CORES = 2
_SUB = 16
_NW = _CORES * _SUB
_RPS = 632
_NPAD = _SUB * _RPS
_CHUNK = 128
_BLKC = 16
_NBLK = 5
_CPS = _BLKC * _NBLK
_EPW = _CPS * _CHUNK
_EPAD = _NW * _EPW

_sc_mesh = plsc.VectorSubcoreMesh(core_axis_name="c", subcore_axis_name="s")


@functools.partial(
    pl.kernel,
    out_type=jax.ShapeDtypeStruct((_CORES, _NPAD, _F), jnp.float32),
    mesh=_sc_mesh,
    scratch_types=[
        pltpu.VMEM_SHARED((_NPAD, _F), jnp.float32),
        pltpu.VMEM((_BLKC, _CHUNK), jnp.int32),
        pltpu.VMEM((_BLKC, _CHUNK), jnp.int32),
        pltpu.VMEM((_CHUNK, _F), jnp.float32),
        pltpu.SemaphoreType.DMA,
    ],
)
def _sc_aggregate(h_hbm, src_hbm, dst_hbm, out_hbm, acc_sh, src_v, dst_v, rows_v, sem):
    c = lax.axis_index("c")
    s = lax.axis_index("s")
    r0 = s * _RPS
    pltpu.sync_copy(h_hbm.at[pl.ds(r0, _RPS)], acc_sh.at[pl.ds(r0, _RPS)])
    plsc.subcore_barrier()

    def block(b, carry):
        pltpu.sync_copy(src_hbm.at[c, s, pl.ds(b * _BLKC, _BLKC)], src_v)
        pltpu.sync_copy(dst_hbm.at[c, s, pl.ds(b * _BLKC, _BLKC)], dst_v)

        def body(j, carry2):
            pltpu.async_copy(h_hbm.at[src_v.at[j]], rows_v, sem).wait()
            pltpu.sync_copy(rows_v, acc_sh.at[dst_v.at[j]], add=True)
            return carry2

        return lax.fori_loop(0, _BLKC, body, carry)

    lax.fori_loop(0, _NBLK, block, 0)
    plsc.subcore_barrier()
    pltpu.sync_copy(acc_sh.at[pl.ds(r0, _RPS)], out_hbm.at[c, pl.ds(r0, _RPS)])


def _mlp_body(agg_ref, h_ref, w1_ref, b1_ref, w2_ref, sc_ref, sh_ref, out_ref):
    x = agg_ref[0] + agg_ref[1] - h_ref[...]
    h = jnp.dot(x, w1_ref[...], preferred_element_type=jnp.float32) + b1_ref[...]
    h = jnp.maximum(h, 0.0)
    h = jnp.dot(h, w2_ref[...], preferred_element_type=jnp.float32)
    out_ref[...] = jnp.maximum(h * sc_ref[...] + sh_ref[...], 0.0)


_tc_mlp = pl.pallas_call(
    _mlp_body,
    out_shape=jax.ShapeDtypeStruct((_NPAD, _F), jnp.float32),
)


def _final_body(agg_ref, h_ref, w1_ref, b1_ref, w2_ref, sc_ref, sh_ref, batch_ref,
                fcw_ref, fcb_ref, out_ref):
    x = agg_ref[0] + agg_ref[1] - h_ref[...]
    h = jnp.dot(x, w1_ref[...], preferred_element_type=jnp.float32) + b1_ref[...]
    h = jnp.maximum(h, 0.0)
    h = jnp.dot(h, w2_ref[...], preferred_element_type=jnp.float32)
    h = jnp.maximum(h * sc_ref[...] + sh_ref[...], 0.0)
    gids = lax.broadcasted_iota(jnp.int32, (_NPAD, _NG), 1)
    onehot = (batch_ref[...] == gids).astype(jnp.float32)
    sums = lax.dot_general(onehot, h, (((0,), (0,)), ((), ())),
                           preferred_element_type=jnp.float32)
    counts = jnp.sum(onehot, axis=0).reshape(_NG, 1)
    pooled = sums / jnp.maximum(counts, 1.0)
    logits = jnp.dot(pooled, fcw_ref[...], preferred_element_type=jnp.float32)
    logits = logits + fcb_ref[...]
    mx = jnp.max(logits, axis=1, keepdims=True)
    lse = jnp.log(jnp.sum(jnp.exp(logits - mx), axis=1, keepdims=True)) + mx
    out_ref[...] = logits - lse


_tc_final = pl.pallas_call(
    _final_body,
    out_shape=jax.ShapeDtypeStruct((_NG, _NCLASS), jnp.float32),
)


def _pad_edges(edge_index):
    pad_e = _EPAD - _E
    spread = _N + (jnp.arange(pad_e, dtype=jnp.int32) % (_NPAD - _N))
    src = jnp.concatenate([edge_index[0], spread])
    dst = jnp.concatenate([edge_index[1], spread])
    return (src.reshape(_CORES, _SUB, _CPS, _CHUNK),
            dst.reshape(_CORES, _SUB, _CPS, _CHUNK))


def kernel(x, edge_index, batch, params, fc_W, fc_b):
    h = jnp.pad(x, ((0, _NPAD - _N), (0, 0)))
    src, dst = _pad_edges(edge_index)
    batch_pad = jnp.concatenate(
        [batch, jnp.full((_NPAD - _N,), _NG, jnp.int32)]).reshape(_NPAD, 1)

    out = None
    for i, (W1, b1, W2, b2, g, be, m, v) in enumerate(params):
        scale = g / jnp.sqrt(v + 1e-5)
        shift = (b2 - m) * scale + be
        aggs = _sc_aggregate(h, src, dst)
        args = (aggs, h, W1, b1.reshape(1, _F), W2,
                scale.reshape(1, _F), shift.reshape(1, _F))
        if i < len(params) - 1:
            h = _tc_mlp(*args)
        else:
            out = _tc_final(*args, batch_pad, fc_W, fc_b.reshape(1, _NCLASS))
    return out


# speedup vs baseline: 11.6747x; 1.5430x over previous
import functools

import jax
import jax.numpy as jnp
from jax import lax
from jax.experimental import pallas as pl
from jax.experimental.pallas import tpu as pltpu
from jax.experimental.pallas import tpu_sc as plsc

_N = 10000
_E = 320000
_F = 128
_NG = 64
_NCLASS = 10

_CORES = 2
_SUB = 16
_NW = _CORES * _SUB
_RPS = 632
_NPAD = _SUB * _RPS
_CHUNK = 128
_BLKC = 40
_NBLK = 2
_CPS = _BLKC * _NBLK
_EPW = _CPS * _CHUNK
_EPAD = _NW * _EPW

_sc_mesh = plsc.VectorSubcoreMesh(core_axis_name="c", subcore_axis_name="s")


@functools.partial(
    pl.kernel,
    out_type=jax.ShapeDtypeStruct((_CORES, _NPAD, _F), jnp.float32),
    mesh=_sc_mesh,
    scratch_types=[
        pltpu.VMEM_SHARED((_NPAD, _F), jnp.float32),
        pltpu.VMEM((_BLKC, _CHUNK), jnp.int32),
        pltpu.VMEM((_BLKC, _CHUNK), jnp.int32),
        pltpu.VMEM((_CHUNK, _F), jnp.float32),
        pltpu.VMEM((_CHUNK, _F), jnp.float32),
        pltpu.SemaphoreType.DMA,
        pltpu.SemaphoreType.DMA,
    ],
)
def _sc_aggregate(h_hbm, src_hbm, dst_hbm, out_hbm, acc_sh, src_v, dst_v,
                  rows0, rows1, sem0, sem1):
    c = lax.axis_index("c")
    s = lax.axis_index("s")
    r0 = s * _RPS
    pltpu.sync_copy(h_hbm.at[pl.ds(r0, _RPS)], acc_sh.at[pl.ds(r0, _RPS)])
    plsc.subcore_barrier()

    def block(b, carry):
        pltpu.sync_copy(src_hbm.at[c, s, pl.ds(b * _BLKC, _BLKC)], src_v)
        pltpu.sync_copy(dst_hbm.at[c, s, pl.ds(b * _BLKC, _BLKC)], dst_v)
        pltpu.async_copy(h_hbm.at[src_v.at[0]], rows0, sem0)
        pltpu.async_copy(h_hbm.at[src_v.at[1]], rows1, sem1)

        def pair(i, carry2):
            j = 2 * i
            pltpu.make_async_copy(h_hbm.at[src_v.at[j]], rows0, sem0).wait()
            pltpu.sync_copy(rows0, acc_sh.at[dst_v.at[j]], add=True)
            pltpu.async_copy(h_hbm.at[src_v.at[j + 2]], rows0, sem0)
            pltpu.make_async_copy(h_hbm.at[src_v.at[j + 1]], rows1, sem1).wait()
            pltpu.sync_copy(rows1, acc_sh.at[dst_v.at[j + 1]], add=True)
            pltpu.async_copy(h_hbm.at[src_v.at[j + 3]], rows1, sem1)
            return carry2

        lax.fori_loop(0, (_BLKC - 2) // 2, pair, carry)
        pltpu.make_async_copy(h_hbm.at[src_v.at[_BLKC - 2]], rows0, sem0).wait()
        pltpu.sync_copy(rows0, acc_sh.at[dst_v.at[_BLKC - 2]], add=True)
        pltpu.make_async_copy(h_hbm.at[src_v.at[_BLKC - 1]], rows1, sem1).wait()
        pltpu.sync_copy(rows1, acc_sh.at[dst_v.at[_BLKC - 1]], add=True)
        return carry

    lax.fori_loop(0, _NBLK, block, 0)
    plsc.subcore_barrier()
    pltpu.sync_copy(acc_sh.at[pl.ds(r0, _RPS)], out_hbm.at[c, pl.ds(r0, _RPS)])


def _mlp_body(agg_ref, h_ref, w1_ref, b1_ref, w2_ref, sc_ref, sh_ref, out_ref):
    x = agg_ref[0] + agg_ref[1] - h_ref[...]
    h = jnp.dot(x, w1_ref[...], preferred_element_type=jnp.float32) + b1_ref[...]
    h = jnp.maximum(h, 0.0)
    h = jnp.dot(h, w2_ref[...], preferred_element_type=jnp.float32)
    out_ref[...] = jnp.maximum(h * sc_ref[...] + sh_ref[...], 0.0)


_tc_mlp = pl.pallas_call(
    _mlp_body,
    out_shape=jax.ShapeDtypeStruct((_NPAD, _F), jnp.float32),
)


def _final_body(agg_ref, h_ref, w1_ref, b1_ref, w2_ref, sc_ref, sh_ref, batch_ref,
                fcw_ref, fcb_ref, out_ref):
    x = agg_ref[0] + agg_ref[1] - h_ref[...]
    h = jnp.dot(x, w1_ref[...], preferred_element_type=jnp.float32) + b1_ref[...]
    h = jnp.maximum(h, 0.0)
    h = jnp.dot(h, w2_ref[...], preferred_element_type=jnp.float32)
    h = jnp.maximum(h * sc_ref[...] + sh_ref[...], 0.0)
    gids = lax.broadcasted_iota(jnp.int32, (_NPAD, _NG), 1)
    onehot = (batch_ref[...] == gids).astype(jnp.float32)
    sums = lax.dot_general(onehot, h, (((0,), (0,)), ((), ())),
                           preferred_element_type=jnp.float32)
    counts = jnp.sum(onehot, axis=0).reshape(_NG, 1)
    pooled = sums / jnp.maximum(counts, 1.0)
    logits = jnp.dot(pooled, fcw_ref[...], preferred_element_type=jnp.float32)
    logits = logits + fcb_ref[...]
    mx = jnp.max(logits, axis=1, keepdims=True)
    lse = jnp.log(jnp.sum(jnp.exp(logits - mx), axis=1, keepdims=True)) + mx
    out_ref[...] = logits - lse


_tc_final = pl.pallas_call(
    _final_body,
    out_shape=jax.ShapeDtypeStruct((_NG, _NCLASS), jnp.float32),
)


def _pad_edges(edge_index):
    pad_e = _EPAD - _E
    spread = _N + (jnp.arange(pad_e, dtype=jnp.int32) % (_NPAD - _N))
    src = jnp.concatenate([edge_index[0], spread])
    dst = jnp.concatenate([edge_index[1], spread])
    return (src.reshape(_CORES, _SUB, _CPS, _CHUNK),
            dst.reshape(_CORES, _SUB, _CPS, _CHUNK))


def kernel(x, edge_index, batch, params, fc_W, fc_b):
    h = jnp.pad(x, ((0, _NPAD - _N), (0, 0)))
    src, dst = _pad_edges(edge_index)
    batch_pad = jnp.concatenate(
        [batch, jnp.full((_NPAD - _N,), _NG, jnp.int32)]).reshape(_NPAD, 1)

    out = None
    for i, (W1, b1, W2, b2, g, be, m, v) in enumerate(params):
        scale = g / jnp.sqrt(v + 1e-5)
        shift = (b2 - m) * scale + be
        aggs = _sc_aggregate(h, src, dst)
        args = (aggs, h, W1, b1.reshape(1, _F), W2,
                scale.reshape(1, _F), shift.reshape(1, _F))
        if i < len(params) - 1:
            h = _tc_mlp(*args)
        else:
            out = _tc_final(*args, batch_pad, fc_W, fc_b.reshape(1, _NCLASS))
    return out
